# baseline (speedup 1.0000x reference)
import functools

import jax
import jax.numpy as jnp
from jax import lax
from jax.experimental import pallas as pl
from jax.experimental.pallas import tpu as pltpu
from jax.experimental.pallas import tpu_sc as plsc

N = 10000
NP = 10240
E = 320000
EP = 327680
ER = EP // 128
G = 64
H = 128
NC, NS = 2, 16
ROWS_T = NP // NS
BR = 1024
NB = NP // BR

NEG = float("-inf")


def _sc_mesh():
    return plsc.VectorSubcoreMesh(
        core_axis_name="c", subcore_axis_name="s",
        num_cores=NC, num_subcores=NS)


def _deg_body(dst_h, w_h, zeros_h, out_h, deg_sp, didx_v, w_v, rows_v):
    c = lax.axis_index("c")
    s = lax.axis_index("s")
    r0 = s * ROWS_T

    def z_loop(q, _):
        r = r0 + q * 128
        pltpu.sync_copy(zeros_h.at[pl.ds(r, 128)], rows_v)
        pltpu.sync_copy(rows_v, deg_sp.at[pl.ds(r, 128)])
        return 0

    lax.fori_loop(0, ROWS_T // 128, z_loop, 0)
    plsc.subcore_barrier()

    rows_per_tile = ER // (NC * NS)
    base = (c * NS + s) * rows_per_tile
    ones16 = jnp.ones((16,), jnp.float32)

    def chunk(i, _):
        r = base + i
        pltpu.sync_copy(dst_h.at[r], didx_v)
        pltpu.sync_copy(w_h.at[r], w_v)
        for jj in range(8):
            w16 = w_v[pl.ds(jj * 16, 16)]
            for ei in range(16):
                rows_v[jj * 16 + ei, pl.ds(0, 16)] = ones16 * w16[ei]
        pltpu.sync_copy(rows_v, deg_sp.at[didx_v], add=True)
        return 0

    lax.fori_loop(0, rows_per_tile, chunk, 0)
    plsc.subcore_barrier()

    def wb(q, _):
        r = r0 + q * 128
        pltpu.sync_copy(deg_sp.at[pl.ds(r, 128)], rows_v)
        pltpu.sync_copy(rows_v, out_h.at[c, pl.ds(r, 128)])
        return 0

    lax.fori_loop(0, ROWS_T // 128, wb, 0)


@functools.cache
def _deg_call():
    return pl.kernel(
        _deg_body,
        out_type=jax.ShapeDtypeStruct((NC, NP, H), jnp.float32),
        mesh=_sc_mesh(),
        scratch_types=[
            pltpu.VMEM_SHARED((NP, H), jnp.float32),
            pltpu.VMEM((128,), jnp.int32),
            pltpu.VMEM((128,), jnp.float32),
            pltpu.VMEM((128, H), jnp.float32),
        ],
    )


def _deg_kernel(dst2, w2, zeros_np):
    return _deg_call()(dst2, w2, zeros_np)


def _edge_body(src_h, dst_h, w_h, xs_h, zeros_h, acc_h, acc_sp, sidx_b,
               didx_b, w_b, rows_a, rows_b, sem_a, sem_b, sem_a2, sem_b2):
    c = lax.axis_index("c")
    s = lax.axis_index("s")
    r0 = s * ROWS_T
    own = ((c == 0) & (s < NS // 2)) | ((c == 1) & (s >= NS // 2))

    def _stage(src_ref):
        def q_loop(q, _):
            r = r0 + q * 128
            pltpu.sync_copy(src_ref.at[pl.ds(r, 128)], rows_a)
            pltpu.sync_copy(rows_a, acc_sp.at[pl.ds(r, 128)])
            return 0
        lax.fori_loop(0, ROWS_T // 128, q_loop, 0)

    @pl.when(own)
    def _():
        _stage(xs_h)

    @pl.when(jnp.logical_not(own))
    def _():
        _stage(zeros_h)

    plsc.subcore_barrier()

    rows_per_tile = ER // (NC * NS)
    base = c * (ER // NC) + s * rows_per_tile
    BT = 16

    def scale(rows, j):
        def sc(j16, _):
            w16 = w_b[j, pl.ds(j16 * 16, 16)]
            for ei in range(16):
                wsc = w16[ei]
                e = j16 * 16 + ei
                for k in range(8):
                    rows[e, pl.ds(k * 16, 16)] = (
                        rows[e, pl.ds(k * 16, 16)] * wsc)
            return 0
        lax.fori_loop(0, 8, sc, 0)

    def wait_gather(rows, sem, j):
        pltpu.make_async_copy(xs_h.at[sidx_b.at[j]], rows, sem).wait()

    def wait_scatter(rows, sem, j):
        pltpu.make_async_copy(rows, acc_sp.at[didx_b.at[j]], sem).wait()

    def batch_body(bb, _):
        rb = base + bb * BT
        pltpu.sync_copy(src_h.at[pl.ds(rb, BT)], sidx_b)
        pltpu.sync_copy(dst_h.at[pl.ds(rb, BT)], didx_b)
        pltpu.sync_copy(w_h.at[pl.ds(rb, BT)], w_b)
        pltpu.async_copy(xs_h.at[sidx_b.at[0]], rows_a, sem_a)

        def pair(j2, _):
            j = j2 * 2

            @pl.when(j2 > 0)
            def _():
                wait_scatter(rows_b, sem_b2, j - 1)

            pltpu.async_copy(xs_h.at[sidx_b.at[j + 1]], rows_b, sem_b)
            wait_gather(rows_a, sem_a, j)
            scale(rows_a, j)
            pltpu.async_copy(rows_a, acc_sp.at[didx_b.at[j]], sem_a2,
                             add=True)
            wait_gather(rows_b, sem_b, j + 1)
            scale(rows_b, j + 1)

            @pl.when(j2 < BT // 2 - 1)
            def _():
                wait_scatter(rows_a, sem_a2, j)
                pltpu.async_copy(xs_h.at[sidx_b.at[j + 2]], rows_a, sem_a)

            pltpu.async_copy(rows_b, acc_sp.at[didx_b.at[j + 1]], sem_b2,
                             add=True)
            return 0

        lax.fori_loop(0, BT // 2, pair, 0)
        wait_scatter(rows_a, sem_a2, BT - 2)
        wait_scatter(rows_b, sem_b2, BT - 1)
        return 0

    lax.fori_loop(0, rows_per_tile // BT, batch_body, 0)
    plsc.subcore_barrier()

    def wb(q, _):
        r = r0 + q * 128
        pltpu.sync_copy(acc_sp.at[pl.ds(r, 128)], rows_a)
        pltpu.sync_copy(rows_a, acc_h.at[c, pl.ds(r, 128)])
        return 0

    lax.fori_loop(0, ROWS_T // 128, wb, 0)


@functools.cache
def _edge_call():
    return pl.kernel(
        _edge_body,
        out_type=jax.ShapeDtypeStruct((NC, NP, H), jnp.float32),
        mesh=_sc_mesh(),
        scratch_types=[
            pltpu.VMEM_SHARED((NP, H), jnp.float32),
            pltpu.VMEM((16, 128), jnp.int32),
            pltpu.VMEM((16, 128), jnp.int32),
            pltpu.VMEM((16, 128), jnp.float32),
            pltpu.VMEM((128, H), jnp.float32),
            pltpu.VMEM((128, H), jnp.float32),
            pltpu.SemaphoreType.DMA,
            pltpu.SemaphoreType.DMA,
            pltpu.SemaphoreType.DMA,
            pltpu.SemaphoreType.DMA,
        ],
    )


def _edge_kernel(src2, dst2, w2, xs, zeros_np):
    return _edge_call()(src2, dst2, w2, xs, zeros_np)


def _pre_body(degp_ref, x_ref, w0_ref, dis_ref, xs_ref):
    deg = 1.0 + degp_ref[0, :, 0:1] + degp_ref[1, :, 0:1]
    dis = lax.rsqrt(deg)
    xw = jnp.dot(x_ref[...], w0_ref[...], preferred_element_type=jnp.float32)
    dis_ref[...] = dis
    xs_ref[...] = dis * xw


def _tc_pre(degp, x, W0):
    return pl.pallas_call(
        _pre_body,
        grid=(NB,),
        in_specs=[
            pl.BlockSpec((NC, BR, H), lambda i: (0, i, 0)),
            pl.BlockSpec((BR, H), lambda i: (i, 0)),
            pl.BlockSpec((H, H), lambda i: (0, 0)),
        ],
        out_specs=[
            pl.BlockSpec((BR, 1), lambda i: (i, 0)),
            pl.BlockSpec((BR, H), lambda i: (i, 0)),
        ],
        out_shape=[
            jax.ShapeDtypeStruct((NP, 1), jnp.float32),
            jax.ShapeDtypeStruct((NP, H), jnp.float32),
        ],
    )(degp, x, W0)


def _stats_body(accp_ref, dis_ref, b_ref, batch_ref, conv_ref,
                ssum_ref, ssum2_ref, cnt_ref):
    pid = pl.program_id(0)
    acc = accp_ref[0] + accp_ref[1]
    conv = acc * dis_ref[...] + b_ref[...]
    conv_ref[...] = conv
    iota = lax.broadcasted_iota(jnp.int32, (1, G), 1)
    S = (batch_ref[...] == iota).astype(jnp.float32)
    dn = (((0,), (0,)), ((), ()))
    ps = lax.dot_general(S, conv, dn, preferred_element_type=jnp.float32)
    ps2 = lax.dot_general(S, conv * conv, dn, preferred_element_type=jnp.float32)
    ones = jnp.ones((S.shape[0], 1), jnp.float32)
    pc = lax.dot_general(S, ones, dn, preferred_element_type=jnp.float32)

    @pl.when(pid == 0)
    def _():
        ssum_ref[...] = ps
        ssum2_ref[...] = ps2
        cnt_ref[...] = pc

    @pl.when(pid != 0)
    def _():
        ssum_ref[...] += ps
        ssum2_ref[...] += ps2
        cnt_ref[...] += pc


def _tc_stats(accp, dis, b, batch2):
    return pl.pallas_call(
        _stats_body,
        grid=(NB,),
        in_specs=[
            pl.BlockSpec((NC, BR, H), lambda i: (0, i, 0)),
            pl.BlockSpec((BR, 1), lambda i: (i, 0)),
            pl.BlockSpec((1, H), lambda i: (0, 0)),
            pl.BlockSpec((BR, 1), lambda i: (i, 0)),
        ],
        out_specs=[
            pl.BlockSpec((BR, H), lambda i: (i, 0)),
            pl.BlockSpec((G, H), lambda i: (0, 0)),
            pl.BlockSpec((G, H), lambda i: (0, 0)),
            pl.BlockSpec((G, 1), lambda i: (0, 0)),
        ],
        out_shape=[
            jax.ShapeDtypeStruct((NP, H), jnp.float32),
            jax.ShapeDtypeStruct((G, H), jnp.float32),
            jax.ShapeDtypeStruct((G, H), jnp.float32),
            jax.ShapeDtypeStruct((G, 1), jnp.float32),
        ],
    )(accp, dis, b, batch2)


def _apply_body(residual, last, conv_ref, hprev_ref, batch_ref, dis_ref,
                ssum_ref, ssum2_ref, cnt_ref, g_ref, be_ref, a_ref, wn_ref,
                h_ref, xs_ref, flat_ref):
    pid = pl.program_id(0)
    cnt_c = jnp.maximum(cnt_ref[...], 1.0)
    a = a_ref[...]
    mean = ssum_ref[...] / cnt_c
    var = ssum2_ref[...] / cnt_c - (2.0 * a - a * a) * mean * mean
    istd = lax.rsqrt(var + 1e-5)
    iota = lax.broadcasted_iota(jnp.int32, (1, G), 1)
    batch = batch_ref[...]
    S = (batch == iota).astype(jnp.float32)
    meanx = jnp.dot(S, mean, preferred_element_type=jnp.float32)
    istdx = jnp.dot(S, istd, preferred_element_type=jnp.float32)
    out1 = conv_ref[...] - a * meanx
    hn = g_ref[...] * out1 * istdx + be_ref[...]
    if residual:
        hn = hn + hprev_ref[...]
    h = jnp.maximum(hn, 0.0)
    h_ref[...] = h
    if not last:
        xw = jnp.dot(h, wn_ref[...], preferred_element_type=jnp.float32)
        xs_ref[...] = dis_ref[...] * xw

    glo = jnp.min(batch)
    ghi = jnp.max(batch)
    gcol = lax.broadcasted_iota(jnp.int32, (G, 1), 0)

    def fb(gi, pf):
        m = batch == gi
        hv = jnp.where(m, h, NEG)
        mv = jnp.max(hv, axis=0, keepdims=True)
        return jnp.maximum(pf, jnp.where(gcol == gi, mv, NEG))

    pf = lax.fori_loop(glo, ghi + 1, fb, jnp.full((G, H), NEG, jnp.float32))

    @pl.when(pid == 0)
    def _():
        flat_ref[...] = pf

    @pl.when(pid != 0)
    def _():
        flat_ref[...] = jnp.maximum(flat_ref[...], pf)


def _tc_apply(residual, last, conv, hprev, batch2, dis, ssum, ssum2, cnt,
              g, be, a, Wn):
    body = functools.partial(_apply_body, residual, last)
    return pl.pallas_call(
        body,
        grid=(NB,),
        in_specs=[
            pl.BlockSpec((BR, H), lambda i: (i, 0)),
            pl.BlockSpec((BR, H), lambda i: (i, 0)),
            pl.BlockSpec((BR, 1), lambda i: (i, 0)),
            pl.BlockSpec((BR, 1), lambda i: (i, 0)),
            pl.BlockSpec((G, H), lambda i: (0, 0)),
            pl.BlockSpec((G, H), lambda i: (0, 0)),
            pl.BlockSpec((G, 1), lambda i: (0, 0)),
            pl.BlockSpec((1, H), lambda i: (0, 0)),
            pl.BlockSpec((1, H), lambda i: (0, 0)),
            pl.BlockSpec((1, H), lambda i: (0, 0)),
            pl.BlockSpec((H, H), lambda i: (0, 0)),
        ],
        out_specs=[
            pl.BlockSpec((BR, H), lambda i: (i, 0)),
            pl.BlockSpec((BR, H), lambda i: (i, 0)),
            pl.BlockSpec((G, H), lambda i: (0, 0)),
        ],
        out_shape=[
            jax.ShapeDtypeStruct((NP, H), jnp.float32),
            jax.ShapeDtypeStruct((NP, H), jnp.float32),
            jax.ShapeDtypeStruct((G, H), jnp.float32),
        ],
    )(conv, hprev, batch2, dis, ssum, ssum2, cnt, g, be, a, Wn)


def _head_body(f0_ref, f1_ref, f2_ref, wd1_ref, bd1_ref, wd2_ref, bd2_ref,
               o_ref):
    fl = f0_ref[...] + f1_ref[...] + f2_ref[...]
    h1 = jnp.dot(fl, wd1_ref[...], preferred_element_type=jnp.float32)
    h1 = jnp.maximum(h1 + bd1_ref[...], 0.0)
    o_ref[...] = jnp.dot(h1, wd2_ref[...],
                         preferred_element_type=jnp.float32) + bd2_ref[...]


def _tc_head(f0, f1, f2, Wd1, bd1, Wd2p, bd2p):
    return pl.pallas_call(
        _head_body,
        out_shape=jax.ShapeDtypeStruct((G, 128), jnp.float32),
    )(f0, f1, f2, Wd1, bd1, Wd2p, bd2p)


def kernel(inputs, edge_index, batch, edge_weight, W0, b0, g0, be0, a0, W1,
           b1, g1, be1, a1, W2, b2, g2, be2, a2, Wd1, bd1, Wd2, bd2):
    f32, i32 = jnp.float32, jnp.int32
    src = edge_index[0]
    dst = edge_index[1]
    pad_e = EP - E
    src2 = jnp.concatenate([src, jnp.zeros((pad_e,), i32)]).reshape(ER, 128)
    dst2 = jnp.concatenate([dst, jnp.zeros((pad_e,), i32)]).reshape(ER, 128)
    w2 = jnp.concatenate([edge_weight,
                          jnp.zeros((pad_e,), f32)]).reshape(ER, 128)
    xp = jnp.pad(inputs, ((0, NP - N), (0, 0)))
    batch2 = jnp.concatenate([batch,
                              jnp.full((NP - N,), G, i32)]).reshape(NP, 1)
    zeros_np = jnp.zeros((NP, H), f32)

    degp = _deg_kernel(dst2, w2, zeros_np)
    dis, xs = _tc_pre(degp, xp, W0)

    row = lambda v: v.reshape(1, -1)
    flats = []
    h = xp
    params = [(b0, g0, be0, a0, W1, False),
              (b1, g1, be1, a1, W2, True),
              (b2, g2, be2, a2, W2, True)]
    for li, (b, g, be, a, Wn, residual) in enumerate(params):
        accp = _edge_kernel(src2, dst2, w2, xs, zeros_np)
        conv, ssum, ssum2, cnt = _tc_stats(accp, dis, row(b), batch2)
        h, xs, flat = _tc_apply(
            residual, li == 2, conv, h, batch2, dis, ssum, ssum2, cnt,
            row(g), row(be), row(a), Wn)
        flats.append(flat)

    Wd2p = jnp.pad(Wd2, ((0, 0), (0, 128 - Wd2.shape[1])))
    bd2p = jnp.pad(bd2, (0, 128 - bd2.shape[0])).reshape(1, 128)
    out = _tc_head(flats[0], flats[1], flats[2], Wd1, row(bd1), Wd2p, bd2p)
    return out[:, :Wd2.shape[1]]


# speedup vs baseline: 1.0388x; 1.0388x over previous
import functools

import jax
import jax.numpy as jnp
from jax import lax
from jax.experimental import pallas as pl
from jax.experimental.pallas import tpu as pltpu
from jax.experimental.pallas import tpu_sc as plsc

N = 10000
NP = 10240
E = 320000
EP = 327680
ER = EP // 128
G = 64
H = 128
NC, NS = 2, 16
ROWS_T = NP // NS
BR = 1024
NB = NP // BR

NEG = float("-inf")


def _sc_mesh():
    return plsc.VectorSubcoreMesh(
        core_axis_name="c", subcore_axis_name="s",
        num_cores=NC, num_subcores=NS)


def _deg_body(dst_h, w_h, zeros_h, out_h, deg_sp, didx_v, w_v, rows_v):
    c = lax.axis_index("c")
    s = lax.axis_index("s")
    r0 = s * ROWS_T

    def z_loop(q, _):
        r = r0 + q * 128
        pltpu.sync_copy(zeros_h.at[pl.ds(r, 128)], rows_v)
        pltpu.sync_copy(rows_v, deg_sp.at[pl.ds(r, 128)])
        return 0

    lax.fori_loop(0, ROWS_T // 128, z_loop, 0)
    plsc.subcore_barrier()

    rows_per_tile = ER // (NC * NS)
    base = (c * NS + s) * rows_per_tile
    ones16 = jnp.ones((16,), jnp.float32)

    def chunk(i, _):
        r = base + i
        pltpu.sync_copy(dst_h.at[r], didx_v)
        pltpu.sync_copy(w_h.at[r], w_v)
        for jj in range(8):
            w16 = w_v[pl.ds(jj * 16, 16)]
            for ei in range(16):
                rows_v[jj * 16 + ei, pl.ds(0, 16)] = ones16 * w16[ei]
        pltpu.sync_copy(rows_v, deg_sp.at[didx_v], add=True)
        return 0

    lax.fori_loop(0, rows_per_tile, chunk, 0)
    plsc.subcore_barrier()

    def wb(q, _):
        r = r0 + q * 128
        pltpu.sync_copy(deg_sp.at[pl.ds(r, 128)], rows_v)
        pltpu.sync_copy(rows_v, out_h.at[c, pl.ds(r, 128)])
        return 0

    lax.fori_loop(0, ROWS_T // 128, wb, 0)


@functools.cache
def _deg_call():
    return pl.kernel(
        _deg_body,
        out_type=jax.ShapeDtypeStruct((NC, NP, H), jnp.float32),
        mesh=_sc_mesh(),
        scratch_types=[
            pltpu.VMEM_SHARED((NP, H), jnp.float32),
            pltpu.VMEM((128,), jnp.int32),
            pltpu.VMEM((128,), jnp.float32),
            pltpu.VMEM((128, H), jnp.float32),
        ],
    )


def _deg_kernel(dst2, w2, zeros_np):
    return _deg_call()(dst2, w2, zeros_np)


def _edge_body(src_h, dst_h, w_h, xs_h, zeros_h, acc_h, acc_sp, sidx_b,
               didx_b, w_b, rows_a, rows_b, sem_a, sem_b, sem_a2, sem_b2):
    c = lax.axis_index("c")
    s = lax.axis_index("s")
    r0 = s * ROWS_T
    own = ((c == 0) & (s < NS // 2)) | ((c == 1) & (s >= NS // 2))

    def _stage(src_ref):
        def q_loop(q, _):
            r = r0 + q * 128
            pltpu.sync_copy(src_ref.at[pl.ds(r, 128)], rows_a)
            pltpu.sync_copy(rows_a, acc_sp.at[pl.ds(r, 128)])
            return 0
        lax.fori_loop(0, ROWS_T // 128, q_loop, 0)

    @pl.when(own)
    def _():
        _stage(xs_h)

    @pl.when(jnp.logical_not(own))
    def _():
        _stage(zeros_h)

    plsc.subcore_barrier()

    rows_per_tile = ER // (NC * NS)
    base = c * (ER // NC) + s * rows_per_tile
    BT = 16

    def scale(rows, j):
        def sc(j16, _):
            w16 = w_b[j, pl.ds(j16 * 16, 16)]
            for ei in range(16):
                wsc = w16[ei]
                e = j16 * 16 + ei
                for k in range(8):
                    rows[e, pl.ds(k * 16, 16)] = (
                        rows[e, pl.ds(k * 16, 16)] * wsc)
            return 0
        lax.fori_loop(0, 8, sc, 0)

    def gstart(rows, sem, j):
        pltpu.async_copy(xs_h.at[sidx_b.at[j, pl.ds(0, 64)]],
                         rows.at[pl.ds(0, 64)], sem)
        pltpu.async_copy(xs_h.at[sidx_b.at[j, pl.ds(64, 64)]],
                         rows.at[pl.ds(64, 64)], sem)

    def finish(rows, sem, j):
        pltpu.make_async_copy(xs_h.at[sidx_b.at[j, pl.ds(0, 64)]],
                              rows.at[pl.ds(0, 64)], sem).wait()
        pltpu.make_async_copy(xs_h.at[sidx_b.at[j, pl.ds(64, 64)]],
                              rows.at[pl.ds(64, 64)], sem).wait()
        scale(rows, j)
        pltpu.sync_copy(rows, acc_sp.at[didx_b.at[j]], add=True)

    def batch_body(bb, _):
        rb = base + bb * BT
        pltpu.sync_copy(src_h.at[pl.ds(rb, BT)], sidx_b)
        pltpu.sync_copy(dst_h.at[pl.ds(rb, BT)], didx_b)
        pltpu.sync_copy(w_h.at[pl.ds(rb, BT)], w_b)
        gstart(rows_a, sem_a, 0)

        def pair(j2, _):
            j = j2 * 2
            gstart(rows_b, sem_b, j + 1)
            finish(rows_a, sem_a, j)

            @pl.when(j2 < BT // 2 - 1)
            def _():
                gstart(rows_a, sem_a, j + 2)

            finish(rows_b, sem_b, j + 1)
            return 0

        lax.fori_loop(0, BT // 2, pair, 0)
        return 0

    lax.fori_loop(0, rows_per_tile // BT, batch_body, 0)
    plsc.subcore_barrier()

    def wb(q, _):
        r = r0 + q * 128
        pltpu.sync_copy(acc_sp.at[pl.ds(r, 128)], rows_a)
        pltpu.sync_copy(rows_a, acc_h.at[c, pl.ds(r, 128)])
        return 0

    lax.fori_loop(0, ROWS_T // 128, wb, 0)


@functools.cache
def _edge_call():
    return pl.kernel(
        _edge_body,
        out_type=jax.ShapeDtypeStruct((NC, NP, H), jnp.float32),
        mesh=_sc_mesh(),
        scratch_types=[
            pltpu.VMEM_SHARED((NP, H), jnp.float32),
            pltpu.VMEM((16, 128), jnp.int32),
            pltpu.VMEM((16, 128), jnp.int32),
            pltpu.VMEM((16, 128), jnp.float32),
            pltpu.VMEM((128, H), jnp.float32),
            pltpu.VMEM((128, H), jnp.float32),
            pltpu.SemaphoreType.DMA,
            pltpu.SemaphoreType.DMA,
            pltpu.SemaphoreType.DMA,
            pltpu.SemaphoreType.DMA,
        ],
    )


def _edge_kernel(src2, dst2, w2, xs, zeros_np):
    return _edge_call()(src2, dst2, w2, xs, zeros_np)


def _pre_body(degp_ref, x_ref, w0_ref, dis_ref, xs_ref):
    deg = 1.0 + degp_ref[0, :, 0:1] + degp_ref[1, :, 0:1]
    dis = lax.rsqrt(deg)
    xw = jnp.dot(x_ref[...], w0_ref[...], preferred_element_type=jnp.float32)
    dis_ref[...] = dis
    xs_ref[...] = dis * xw


def _tc_pre(degp, x, W0):
    return pl.pallas_call(
        _pre_body,
        grid=(NB,),
        in_specs=[
            pl.BlockSpec((NC, BR, H), lambda i: (0, i, 0)),
            pl.BlockSpec((BR, H), lambda i: (i, 0)),
            pl.BlockSpec((H, H), lambda i: (0, 0)),
        ],
        out_specs=[
            pl.BlockSpec((BR, 1), lambda i: (i, 0)),
            pl.BlockSpec((BR, H), lambda i: (i, 0)),
        ],
        out_shape=[
            jax.ShapeDtypeStruct((NP, 1), jnp.float32),
            jax.ShapeDtypeStruct((NP, H), jnp.float32),
        ],
    )(degp, x, W0)


def _stats_body(accp_ref, dis_ref, b_ref, batch_ref, conv_ref,
                ssum_ref, ssum2_ref, cnt_ref):
    pid = pl.program_id(0)
    acc = accp_ref[0] + accp_ref[1]
    conv = acc * dis_ref[...] + b_ref[...]
    conv_ref[...] = conv
    iota = lax.broadcasted_iota(jnp.int32, (1, G), 1)
    S = (batch_ref[...] == iota).astype(jnp.float32)
    dn = (((0,), (0,)), ((), ()))
    ps = lax.dot_general(S, conv, dn, preferred_element_type=jnp.float32)
    ps2 = lax.dot_general(S, conv * conv, dn, preferred_element_type=jnp.float32)
    ones = jnp.ones((S.shape[0], 1), jnp.float32)
    pc = lax.dot_general(S, ones, dn, preferred_element_type=jnp.float32)

    @pl.when(pid == 0)
    def _():
        ssum_ref[...] = ps
        ssum2_ref[...] = ps2
        cnt_ref[...] = pc

    @pl.when(pid != 0)
    def _():
        ssum_ref[...] += ps
        ssum2_ref[...] += ps2
        cnt_ref[...] += pc


def _tc_stats(accp, dis, b, batch2):
    return pl.pallas_call(
        _stats_body,
        grid=(NB,),
        in_specs=[
            pl.BlockSpec((NC, BR, H), lambda i: (0, i, 0)),
            pl.BlockSpec((BR, 1), lambda i: (i, 0)),
            pl.BlockSpec((1, H), lambda i: (0, 0)),
            pl.BlockSpec((BR, 1), lambda i: (i, 0)),
        ],
        out_specs=[
            pl.BlockSpec((BR, H), lambda i: (i, 0)),
            pl.BlockSpec((G, H), lambda i: (0, 0)),
            pl.BlockSpec((G, H), lambda i: (0, 0)),
            pl.BlockSpec((G, 1), lambda i: (0, 0)),
        ],
        out_shape=[
            jax.ShapeDtypeStruct((NP, H), jnp.float32),
            jax.ShapeDtypeStruct((G, H), jnp.float32),
            jax.ShapeDtypeStruct((G, H), jnp.float32),
            jax.ShapeDtypeStruct((G, 1), jnp.float32),
        ],
    )(accp, dis, b, batch2)


def _apply_body(residual, last, conv_ref, hprev_ref, batch_ref, dis_ref,
                ssum_ref, ssum2_ref, cnt_ref, g_ref, be_ref, a_ref, wn_ref,
                h_ref, xs_ref, flat_ref):
    pid = pl.program_id(0)
    cnt_c = jnp.maximum(cnt_ref[...], 1.0)
    a = a_ref[...]
    mean = ssum_ref[...] / cnt_c
    var = ssum2_ref[...] / cnt_c - (2.0 * a - a * a) * mean * mean
    istd = lax.rsqrt(var + 1e-5)
    iota = lax.broadcasted_iota(jnp.int32, (1, G), 1)
    batch = batch_ref[...]
    S = (batch == iota).astype(jnp.float32)
    meanx = jnp.dot(S, mean, preferred_element_type=jnp.float32)
    istdx = jnp.dot(S, istd, preferred_element_type=jnp.float32)
    out1 = conv_ref[...] - a * meanx
    hn = g_ref[...] * out1 * istdx + be_ref[...]
    if residual:
        hn = hn + hprev_ref[...]
    h = jnp.maximum(hn, 0.0)
    h_ref[...] = h
    if not last:
        xw = jnp.dot(h, wn_ref[...], preferred_element_type=jnp.float32)
        xs_ref[...] = dis_ref[...] * xw

    glo = jnp.min(batch)
    ghi = jnp.max(batch)
    gcol = lax.broadcasted_iota(jnp.int32, (G, 1), 0)

    def fb(gi, pf):
        m = batch == gi
        hv = jnp.where(m, h, NEG)
        mv = jnp.max(hv, axis=0, keepdims=True)
        return jnp.maximum(pf, jnp.where(gcol == gi, mv, NEG))

    pf = lax.fori_loop(glo, ghi + 1, fb, jnp.full((G, H), NEG, jnp.float32))

    @pl.when(pid == 0)
    def _():
        flat_ref[...] = pf

    @pl.when(pid != 0)
    def _():
        flat_ref[...] = jnp.maximum(flat_ref[...], pf)


def _tc_apply(residual, last, conv, hprev, batch2, dis, ssum, ssum2, cnt,
              g, be, a, Wn):
    body = functools.partial(_apply_body, residual, last)
    return pl.pallas_call(
        body,
        grid=(NB,),
        in_specs=[
            pl.BlockSpec((BR, H), lambda i: (i, 0)),
            pl.BlockSpec((BR, H), lambda i: (i, 0)),
            pl.BlockSpec((BR, 1), lambda i: (i, 0)),
            pl.BlockSpec((BR, 1), lambda i: (i, 0)),
            pl.BlockSpec((G, H), lambda i: (0, 0)),
            pl.BlockSpec((G, H), lambda i: (0, 0)),
            pl.BlockSpec((G, 1), lambda i: (0, 0)),
            pl.BlockSpec((1, H), lambda i: (0, 0)),
            pl.BlockSpec((1, H), lambda i: (0, 0)),
            pl.BlockSpec((1, H), lambda i: (0, 0)),
            pl.BlockSpec((H, H), lambda i: (0, 0)),
        ],
        out_specs=[
            pl.BlockSpec((BR, H), lambda i: (i, 0)),
            pl.BlockSpec((BR, H), lambda i: (i, 0)),
            pl.BlockSpec((G, H), lambda i: (0, 0)),
        ],
        out_shape=[
            jax.ShapeDtypeStruct((NP, H), jnp.float32),
            jax.ShapeDtypeStruct((NP, H), jnp.float32),
            jax.ShapeDtypeStruct((G, H), jnp.float32),
        ],
    )(conv, hprev, batch2, dis, ssum, ssum2, cnt, g, be, a, Wn)


def _head_body(f0_ref, f1_ref, f2_ref, wd1_ref, bd1_ref, wd2_ref, bd2_ref,
               o_ref):
    fl = f0_ref[...] + f1_ref[...] + f2_ref[...]
    h1 = jnp.dot(fl, wd1_ref[...], preferred_element_type=jnp.float32)
    h1 = jnp.maximum(h1 + bd1_ref[...], 0.0)
    o_ref[...] = jnp.dot(h1, wd2_ref[...],
                         preferred_element_type=jnp.float32) + bd2_ref[...]


def _tc_head(f0, f1, f2, Wd1, bd1, Wd2p, bd2p):
    return pl.pallas_call(
        _head_body,
        out_shape=jax.ShapeDtypeStruct((G, 128), jnp.float32),
    )(f0, f1, f2, Wd1, bd1, Wd2p, bd2p)


def kernel(inputs, edge_index, batch, edge_weight, W0, b0, g0, be0, a0, W1,
           b1, g1, be1, a1, W2, b2, g2, be2, a2, Wd1, bd1, Wd2, bd2):
    f32, i32 = jnp.float32, jnp.int32
    src = edge_index[0]
    dst = edge_index[1]
    pad_e = EP - E
    src2 = jnp.concatenate([src, jnp.zeros((pad_e,), i32)]).reshape(ER, 128)
    dst2 = jnp.concatenate([dst, jnp.zeros((pad_e,), i32)]).reshape(ER, 128)
    w2 = jnp.concatenate([edge_weight,
                          jnp.zeros((pad_e,), f32)]).reshape(ER, 128)
    xp = jnp.pad(inputs, ((0, NP - N), (0, 0)))
    batch2 = jnp.concatenate([batch,
                              jnp.full((NP - N,), G, i32)]).reshape(NP, 1)
    zeros_np = jnp.zeros((NP, H), f32)

    degp = _deg_kernel(dst2, w2, zeros_np)
    dis, xs = _tc_pre(degp, xp, W0)

    row = lambda v: v.reshape(1, -1)
    flats = []
    h = xp
    params = [(b0, g0, be0, a0, W1, False),
              (b1, g1, be1, a1, W2, True),
              (b2, g2, be2, a2, W2, True)]
    for li, (b, g, be, a, Wn, residual) in enumerate(params):
        accp = _edge_kernel(src2, dst2, w2, xs, zeros_np)
        conv, ssum, ssum2, cnt = _tc_stats(accp, dis, row(b), batch2)
        h, xs, flat = _tc_apply(
            residual, li == 2, conv, h, batch2, dis, ssum, ssum2, cnt,
            row(g), row(be), row(a), Wn)
        flats.append(flat)

    Wd2p = jnp.pad(Wd2, ((0, 0), (0, 128 - Wd2.shape[1])))
    bd2p = jnp.pad(bd2, (0, 128 - bd2.shape[0])).reshape(1, 128)
    out = _tc_head(flats[0], flats[1], flats[2], Wd1, row(bd1), Wd2p, bd2p)
    return out[:, :Wd2.shape[1]]


# speedup vs baseline: 1.0810x; 1.0406x over previous
import functools

import jax
import jax.numpy as jnp
from jax import lax
from jax.experimental import pallas as pl
from jax.experimental.pallas import tpu as pltpu
from jax.experimental.pallas import tpu_sc as plsc

N = 10000
NP = 10240
E = 320000
EP = 327680
ER = EP // 128
G = 64
H = 128
NC, NS = 2, 16
ROWS_T = NP // NS
BR = 1024
NB = NP // BR

NEG = float("-inf")


def _sc_mesh():
    return plsc.VectorSubcoreMesh(
        core_axis_name="c", subcore_axis_name="s",
        num_cores=NC, num_subcores=NS)


def _deg_body(dst_h, w_h, zeros_h, out_h, deg_sp, didx_b, w_b, rows_v):
    c = lax.axis_index("c")
    s = lax.axis_index("s")
    r0 = s * ROWS_T

    def z_loop(q, _):
        r = r0 + q * 128
        pltpu.sync_copy(zeros_h.at[pl.ds(r, 128)], rows_v)
        pltpu.sync_copy(rows_v, deg_sp.at[pl.ds(r, 128)])
        return 0

    lax.fori_loop(0, ROWS_T // 128, z_loop, 0)
    plsc.subcore_barrier()

    rows_per_tile = ER // (NC * NS)
    base = (c * NS + s) * rows_per_tile
    ones16 = jnp.ones((16,), jnp.float32)
    BT = 16

    def batchb(bb, _):
        rb = base + bb * BT
        pltpu.sync_copy(dst_h.at[pl.ds(rb, BT)], didx_b)
        pltpu.sync_copy(w_h.at[pl.ds(rb, BT)], w_b)

        def chunk(i, _):
            for jj in range(8):
                w16 = w_b[i, pl.ds(jj * 16, 16)]
                for ei in range(16):
                    rows_v[jj * 16 + ei, pl.ds(0, 16)] = ones16 * w16[ei]
            pltpu.sync_copy(rows_v, deg_sp.at[didx_b.at[i]], add=True)
            return 0

        lax.fori_loop(0, BT, chunk, 0)
        return 0

    lax.fori_loop(0, rows_per_tile // BT, batchb, 0)
    plsc.subcore_barrier()

    def wb(q, _):
        r = r0 + q * 128
        pltpu.sync_copy(deg_sp.at[pl.ds(r, 128)], rows_v)
        pltpu.sync_copy(rows_v, out_h.at[c, pl.ds(r, 128)])
        return 0

    lax.fori_loop(0, ROWS_T // 128, wb, 0)


@functools.cache
def _deg_call():
    return pl.kernel(
        _deg_body,
        out_type=jax.ShapeDtypeStruct((NC, NP, H), jnp.float32),
        mesh=_sc_mesh(),
        scratch_types=[
            pltpu.VMEM_SHARED((NP, H), jnp.float32),
            pltpu.VMEM((16, 128), jnp.int32),
            pltpu.VMEM((16, 128), jnp.float32),
            pltpu.VMEM((128, H), jnp.float32),
        ],
    )


def _deg_kernel(dst2, w2, zeros_np):
    return _deg_call()(dst2, w2, zeros_np)


def _edge_body(src_h, dst_h, w_h, xs_h, zeros_h, acc_h, acc_sp, sidx_b,
               didx_b, w_b, rows_a, rows_b, sem_a, sem_b):
    c = lax.axis_index("c")
    s = lax.axis_index("s")
    r0 = s * ROWS_T
    own = ((c == 0) & (s < NS // 2)) | ((c == 1) & (s >= NS // 2))

    def _stage(src_ref):
        def q_loop(q, _):
            r = r0 + q * 128
            pltpu.sync_copy(src_ref.at[pl.ds(r, 128)], rows_a)
            pltpu.sync_copy(rows_a, acc_sp.at[pl.ds(r, 128)])
            return 0
        lax.fori_loop(0, ROWS_T // 128, q_loop, 0)

    @pl.when(own)
    def _():
        _stage(xs_h)

    @pl.when(jnp.logical_not(own))
    def _():
        _stage(zeros_h)

    plsc.subcore_barrier()

    rows_per_tile = ER // (NC * NS)
    base = c * (ER // NC) + s * rows_per_tile
    BT = 16

    def scale(rows, j):
        def sc(j16, _):
            w16 = w_b[j, pl.ds(j16 * 16, 16)]
            for ei in range(16):
                wsc = w16[ei]
                e = j16 * 16 + ei
                for k in range(8):
                    rows[e, pl.ds(k * 16, 16)] = (
                        rows[e, pl.ds(k * 16, 16)] * wsc)
            return 0
        lax.fori_loop(0, 8, sc, 0)

    def finish(rows, sem, j):
        pltpu.make_async_copy(xs_h.at[sidx_b.at[j]], rows, sem).wait()
        scale(rows, j)
        pltpu.sync_copy(rows, acc_sp.at[didx_b.at[j]], add=True)

    def batch_body(bb, _):
        rb = base + bb * BT
        pltpu.sync_copy(src_h.at[pl.ds(rb, BT)], sidx_b)
        pltpu.sync_copy(dst_h.at[pl.ds(rb, BT)], didx_b)
        pltpu.sync_copy(w_h.at[pl.ds(rb, BT)], w_b)
        pltpu.async_copy(xs_h.at[sidx_b.at[0]], rows_a, sem_a)

        def pair(j2, _):
            j = j2 * 2
            pltpu.async_copy(xs_h.at[sidx_b.at[j + 1]], rows_b, sem_b)
            finish(rows_a, sem_a, j)

            @pl.when(j2 < BT // 2 - 1)
            def _():
                pltpu.async_copy(xs_h.at[sidx_b.at[j + 2]], rows_a, sem_a)

            finish(rows_b, sem_b, j + 1)
            return 0

        lax.fori_loop(0, BT // 2, pair, 0)
        return 0

    lax.fori_loop(0, rows_per_tile // BT, batch_body, 0)
    plsc.subcore_barrier()

    def wb(q, _):
        r = r0 + q * 128
        pltpu.sync_copy(acc_sp.at[pl.ds(r, 128)], rows_a)
        pltpu.sync_copy(rows_a, acc_h.at[c, pl.ds(r, 128)])
        return 0

    lax.fori_loop(0, ROWS_T // 128, wb, 0)


@functools.cache
def _edge_call():
    return pl.kernel(
        _edge_body,
        out_type=jax.ShapeDtypeStruct((NC, NP, H), jnp.float32),
        mesh=_sc_mesh(),
        scratch_types=[
            pltpu.VMEM_SHARED((NP, H), jnp.float32),
            pltpu.VMEM((16, 128), jnp.int32),
            pltpu.VMEM((16, 128), jnp.int32),
            pltpu.VMEM((16, 128), jnp.float32),
            pltpu.VMEM((128, H), jnp.float32),
            pltpu.VMEM((128, H), jnp.float32),
            pltpu.SemaphoreType.DMA,
            pltpu.SemaphoreType.DMA,
        ],
    )


def _edge_kernel(src2, dst2, w2, xs, zeros_np):
    return _edge_call()(src2, dst2, w2, xs, zeros_np)


def _pre_body(degp_ref, x_ref, w0_ref, dis_ref, xs_ref):
    deg = 1.0 + degp_ref[0, :, 0:1] + degp_ref[1, :, 0:1]
    dis = lax.rsqrt(deg)
    xw = jnp.dot(x_ref[...], w0_ref[...], preferred_element_type=jnp.float32)
    dis_ref[...] = dis
    xs_ref[...] = dis * xw


def _tc_pre(degp, x, W0):
    return pl.pallas_call(
        _pre_body,
        grid=(NB,),
        in_specs=[
            pl.BlockSpec((NC, BR, H), lambda i: (0, i, 0)),
            pl.BlockSpec((BR, H), lambda i: (i, 0)),
            pl.BlockSpec((H, H), lambda i: (0, 0)),
        ],
        out_specs=[
            pl.BlockSpec((BR, 1), lambda i: (i, 0)),
            pl.BlockSpec((BR, H), lambda i: (i, 0)),
        ],
        out_shape=[
            jax.ShapeDtypeStruct((NP, 1), jnp.float32),
            jax.ShapeDtypeStruct((NP, H), jnp.float32),
        ],
    )(degp, x, W0)


def _stats_body(accp_ref, dis_ref, b_ref, batch_ref, conv_ref,
                ssum_ref, ssum2_ref, cnt_ref):
    pid = pl.program_id(0)
    acc = accp_ref[0] + accp_ref[1]
    conv = acc * dis_ref[...] + b_ref[...]
    conv_ref[...] = conv
    iota = lax.broadcasted_iota(jnp.int32, (1, G), 1)
    S = (batch_ref[...] == iota).astype(jnp.float32)
    dn = (((0,), (0,)), ((), ()))
    ps = lax.dot_general(S, conv, dn, preferred_element_type=jnp.float32)
    ps2 = lax.dot_general(S, conv * conv, dn, preferred_element_type=jnp.float32)
    ones = jnp.ones((S.shape[0], 1), jnp.float32)
    pc = lax.dot_general(S, ones, dn, preferred_element_type=jnp.float32)

    @pl.when(pid == 0)
    def _():
        ssum_ref[...] = ps
        ssum2_ref[...] = ps2
        cnt_ref[...] = pc

    @pl.when(pid != 0)
    def _():
        ssum_ref[...] += ps
        ssum2_ref[...] += ps2
        cnt_ref[...] += pc


def _tc_stats(accp, dis, b, batch2):
    return pl.pallas_call(
        _stats_body,
        grid=(NB,),
        in_specs=[
            pl.BlockSpec((NC, BR, H), lambda i: (0, i, 0)),
            pl.BlockSpec((BR, 1), lambda i: (i, 0)),
            pl.BlockSpec((1, H), lambda i: (0, 0)),
            pl.BlockSpec((BR, 1), lambda i: (i, 0)),
        ],
        out_specs=[
            pl.BlockSpec((BR, H), lambda i: (i, 0)),
            pl.BlockSpec((G, H), lambda i: (0, 0)),
            pl.BlockSpec((G, H), lambda i: (0, 0)),
            pl.BlockSpec((G, 1), lambda i: (0, 0)),
        ],
        out_shape=[
            jax.ShapeDtypeStruct((NP, H), jnp.float32),
            jax.ShapeDtypeStruct((G, H), jnp.float32),
            jax.ShapeDtypeStruct((G, H), jnp.float32),
            jax.ShapeDtypeStruct((G, 1), jnp.float32),
        ],
    )(accp, dis, b, batch2)


def _apply_body(residual, last, conv_ref, hprev_ref, batch_ref, dis_ref,
                ssum_ref, ssum2_ref, cnt_ref, g_ref, be_ref, a_ref, wn_ref,
                h_ref, xs_ref, flat_ref):
    pid = pl.program_id(0)
    cnt_c = jnp.maximum(cnt_ref[...], 1.0)
    a = a_ref[...]
    mean = ssum_ref[...] / cnt_c
    var = ssum2_ref[...] / cnt_c - (2.0 * a - a * a) * mean * mean
    istd = lax.rsqrt(var + 1e-5)
    iota = lax.broadcasted_iota(jnp.int32, (1, G), 1)
    batch = batch_ref[...]
    S = (batch == iota).astype(jnp.float32)
    meanx = jnp.dot(S, mean, preferred_element_type=jnp.float32)
    istdx = jnp.dot(S, istd, preferred_element_type=jnp.float32)
    out1 = conv_ref[...] - a * meanx
    hn = g_ref[...] * out1 * istdx + be_ref[...]
    if residual:
        hn = hn + hprev_ref[...]
    h = jnp.maximum(hn, 0.0)
    h_ref[...] = h
    if not last:
        xw = jnp.dot(h, wn_ref[...], preferred_element_type=jnp.float32)
        xs_ref[...] = dis_ref[...] * xw

    glo = jnp.min(batch)
    ghi = jnp.max(batch)
    gcol = lax.broadcasted_iota(jnp.int32, (G, 1), 0)

    def fb(gi, pf):
        m = batch == gi
        hv = jnp.where(m, h, NEG)
        mv = jnp.max(hv, axis=0, keepdims=True)
        return jnp.maximum(pf, jnp.where(gcol == gi, mv, NEG))

    pf = lax.fori_loop(glo, ghi + 1, fb, jnp.full((G, H), NEG, jnp.float32))

    @pl.when(pid == 0)
    def _():
        flat_ref[...] = pf

    @pl.when(pid != 0)
    def _():
        flat_ref[...] = jnp.maximum(flat_ref[...], pf)


def _tc_apply(residual, last, conv, hprev, batch2, dis, ssum, ssum2, cnt,
              g, be, a, Wn):
    body = functools.partial(_apply_body, residual, last)
    return pl.pallas_call(
        body,
        grid=(NB,),
        in_specs=[
            pl.BlockSpec((BR, H), lambda i: (i, 0)),
            pl.BlockSpec((BR, H), lambda i: (i, 0)),
            pl.BlockSpec((BR, 1), lambda i: (i, 0)),
            pl.BlockSpec((BR, 1), lambda i: (i, 0)),
            pl.BlockSpec((G, H), lambda i: (0, 0)),
            pl.BlockSpec((G, H), lambda i: (0, 0)),
            pl.BlockSpec((G, 1), lambda i: (0, 0)),
            pl.BlockSpec((1, H), lambda i: (0, 0)),
            pl.BlockSpec((1, H), lambda i: (0, 0)),
            pl.BlockSpec((1, H), lambda i: (0, 0)),
            pl.BlockSpec((H, H), lambda i: (0, 0)),
        ],
        out_specs=[
            pl.BlockSpec((BR, H), lambda i: (i, 0)),
            pl.BlockSpec((BR, H), lambda i: (i, 0)),
            pl.BlockSpec((G, H), lambda i: (0, 0)),
        ],
        out_shape=[
            jax.ShapeDtypeStruct((NP, H), jnp.float32),
            jax.ShapeDtypeStruct((NP, H), jnp.float32),
            jax.ShapeDtypeStruct((G, H), jnp.float32),
        ],
    )(conv, hprev, batch2, dis, ssum, ssum2, cnt, g, be, a, Wn)


def _head_body(f0_ref, f1_ref, f2_ref, wd1_ref, bd1_ref, wd2_ref, bd2_ref,
               o_ref):
    fl = f0_ref[...] + f1_ref[...] + f2_ref[...]
    h1 = jnp.dot(fl, wd1_ref[...], preferred_element_type=jnp.float32)
    h1 = jnp.maximum(h1 + bd1_ref[...], 0.0)
    o_ref[...] = jnp.dot(h1, wd2_ref[...],
                         preferred_element_type=jnp.float32) + bd2_ref[...]


def _tc_head(f0, f1, f2, Wd1, bd1, Wd2p, bd2p):
    return pl.pallas_call(
        _head_body,
        out_shape=jax.ShapeDtypeStruct((G, 128), jnp.float32),
    )(f0, f1, f2, Wd1, bd1, Wd2p, bd2p)


def kernel(inputs, edge_index, batch, edge_weight, W0, b0, g0, be0, a0, W1,
           b1, g1, be1, a1, W2, b2, g2, be2, a2, Wd1, bd1, Wd2, bd2):
    f32, i32 = jnp.float32, jnp.int32
    src = edge_index[0]
    dst = edge_index[1]
    pad_e = EP - E
    src2 = jnp.concatenate([src, jnp.zeros((pad_e,), i32)]).reshape(ER, 128)
    dst2 = jnp.concatenate([dst, jnp.zeros((pad_e,), i32)]).reshape(ER, 128)
    w2 = jnp.concatenate([edge_weight,
                          jnp.zeros((pad_e,), f32)]).reshape(ER, 128)
    xp = jnp.pad(inputs, ((0, NP - N), (0, 0)))
    batch2 = jnp.concatenate([batch,
                              jnp.full((NP - N,), G, i32)]).reshape(NP, 1)
    zeros_np = jnp.zeros((NP, H), f32)

    degp = _deg_kernel(dst2, w2, zeros_np)
    dis, xs = _tc_pre(degp, xp, W0)

    row = lambda v: v.reshape(1, -1)
    flats = []
    h = xp
    params = [(b0, g0, be0, a0, W1, False),
              (b1, g1, be1, a1, W2, True),
              (b2, g2, be2, a2, W2, True)]
    for li, (b, g, be, a, Wn, residual) in enumerate(params):
        accp = _edge_kernel(src2, dst2, w2, xs, zeros_np)
        conv, ssum, ssum2, cnt = _tc_stats(accp, dis, row(b), batch2)
        h, xs, flat = _tc_apply(
            residual, li == 2, conv, h, batch2, dis, ssum, ssum2, cnt,
            row(g), row(be), row(a), Wn)
        flats.append(flat)

    Wd2p = jnp.pad(Wd2, ((0, 0), (0, 128 - Wd2.shape[1])))
    bd2p = jnp.pad(bd2, (0, 128 - bd2.shape[0])).reshape(1, 128)
    out = _tc_head(flats[0], flats[1], flats[2], Wd1, row(bd1), Wd2p, bd2p)
    return out[:, :Wd2.shape[1]]
